# baseline (device time: 31682 ns/iter reference)
import jax
import jax.numpy as jnp
from jax import lax
from jax.experimental import pallas as pl
from jax.experimental.pallas import tpu as pltpu

N_DEV = 16
PLANE = 4
ZDIM = 4
M = 768
N = 768
CHUNK = M // N_DEV
RBLOCK = M // PLANE

N_SEND = 30


def kernel(A, B):
    def body(a_ref, b_ref, out_ref, acc_ref, sbuf, prbuf, prb16, pbuf,
             zbuf, gbuf, send_sems, prsems, zrsems, csems):
        my = lax.axis_index("i")
        z = my // PLANE
        p = my % PLANE

        def plane_peer(off):
            return z * PLANE + (p + off) % PLANE

        def col_peer(off):
            return ((z + off) % ZDIM) * PLANE + p

        barrier_sem = pltpu.get_barrier_semaphore()
        for off in range(1, PLANE):
            pl.semaphore_signal(
                barrier_sem, inc=1, device_id=(plane_peer(off),),
                device_id_type=pl.DeviceIdType.MESH,
            )
        for off in range(1, ZDIM):
            pl.semaphore_signal(
                barrier_sem, inc=1, device_id=(col_peer(off),),
                device_id_type=pl.DeviceIdType.MESH,
            )
        pl.semaphore_wait(barrier_sem, 6)

        sends = []
        slot = 0

        zz_order = [1, 2, 3, 0]
        for off in range(1, PLANE):
            pp = (p + off) % PLANE
            sbuf[pl.ds(pp * RBLOCK, RBLOCK), :] = jnp.dot(
                a_ref[pl.ds(pp * RBLOCK, RBLOCK), :], b_ref[...],
                preferred_element_type=jnp.float32,
            ).astype(jnp.bfloat16)
            for zoff in zz_order:
                zz = (z + zoff) % ZDIM
                rdma = pltpu.make_async_remote_copy(
                    src_ref=sbuf.at[
                        pl.ds(pp * RBLOCK + zz * CHUNK, CHUNK), :],
                    dst_ref=pbuf.at[p, pl.ds(zz * CHUNK, CHUNK), :],
                    send_sem=send_sems.at[slot],
                    recv_sem=prsems.at[p * ZDIM + zz],
                    device_id=(plane_peer(off),),
                    device_id_type=pl.DeviceIdType.MESH,
                )
                rdma.start()
                sends.append(rdma)
                slot += 1
        acc_ref[...] = jnp.dot(
            a_ref[pl.ds(p * RBLOCK, RBLOCK), :], b_ref[...],
            preferred_element_type=jnp.float32,
        )

        for zoff in zz_order:
            zz = (z + zoff) % ZDIM
            sub = acc_ref[pl.ds(zz * CHUNK, CHUNK), :]
            for off in range(1, PLANE):
                pp = (p - off) % PLANE
                recv = pltpu.make_async_remote_copy(
                    src_ref=pbuf.at[pp, pl.ds(zz * CHUNK, CHUNK), :],
                    dst_ref=pbuf.at[pp, pl.ds(zz * CHUNK, CHUNK), :],
                    send_sem=send_sems.at[0],
                    recv_sem=prsems.at[pp * ZDIM + zz],
                    device_id=(my,), device_id_type=pl.DeviceIdType.MESH,
                )
                recv.wait_recv()
                sub = sub + pbuf[pp, pl.ds(zz * CHUNK, CHUNK), :].astype(
                    jnp.float32)
            prbuf[pl.ds(zz * CHUNK, CHUNK), :] = sub
            if zoff != 0:
                prb16[pl.ds(zz * CHUNK, CHUNK), :] = sub.astype(jnp.bfloat16)
                rdma = pltpu.make_async_remote_copy(
                    src_ref=prb16.at[pl.ds(zz * CHUNK, CHUNK), :],
                    dst_ref=zbuf.at[z],
                    send_sem=send_sems.at[slot],
                    recv_sem=zrsems.at[z],
                    device_id=(col_peer(zoff),),
                    device_id_type=pl.DeviceIdType.MESH,
                )
                rdma.start()
                sends.append(rdma)
                slot += 1

        red = prbuf[pl.ds(z * CHUNK, CHUNK), :]
        for off in range(1, ZDIM):
            zz = (z - off) % ZDIM
            recv = pltpu.make_async_remote_copy(
                src_ref=zbuf.at[zz], dst_ref=zbuf.at[zz],
                send_sem=send_sems.at[0], recv_sem=zrsems.at[zz],
                device_id=(my,), device_id_type=pl.DeviceIdType.MESH,
            )
            recv.wait_recv()
            red = red + zbuf[zz].astype(jnp.float32)
        c = p * PLANE + z
        out_ref[pl.ds(c * CHUNK, CHUNK), :] = red
        gbuf[c] = red.astype(jnp.bfloat16)

        for off in range(1, ZDIM):
            rdma = pltpu.make_async_remote_copy(
                src_ref=gbuf.at[c], dst_ref=gbuf.at[c],
                send_sem=send_sems.at[slot],
                recv_sem=csems.at[c],
                device_id=(col_peer(off),),
                device_id_type=pl.DeviceIdType.MESH,
            )
            rdma.start()
            sends.append(rdma)
            slot += 1
        for off in range(1, PLANE):
            rdma = pltpu.make_async_remote_copy(
                src_ref=gbuf.at[c], dst_ref=gbuf.at[c],
                send_sem=send_sems.at[slot],
                recv_sem=csems.at[c],
                device_id=(plane_peer(off),),
                device_id_type=pl.DeviceIdType.MESH,
            )
            rdma.start()
            sends.append(rdma)
            slot += 1

        for off in range(1, ZDIM):
            zz = (z - off) % ZDIM
            cc = p * PLANE + zz
            recv = pltpu.make_async_remote_copy(
                src_ref=gbuf.at[cc], dst_ref=gbuf.at[cc],
                send_sem=send_sems.at[0], recv_sem=csems.at[cc],
                device_id=(my,), device_id_type=pl.DeviceIdType.MESH,
            )
            recv.wait_recv()
            for poff in range(1, PLANE):
                rdma = pltpu.make_async_remote_copy(
                    src_ref=gbuf.at[cc], dst_ref=gbuf.at[cc],
                    send_sem=send_sems.at[slot],
                    recv_sem=csems.at[cc],
                    device_id=(plane_peer(poff),),
                    device_id_type=pl.DeviceIdType.MESH,
                )
                rdma.start()
                sends.append(rdma)
                slot += 1
            out_ref[pl.ds(cc * CHUNK, CHUNK), :] = gbuf[cc].astype(
                jnp.float32)

        for off in range(1, PLANE):
            pp = (p - off) % PLANE
            for zz in range(ZDIM):
                cc = pp * PLANE + zz
                recv = pltpu.make_async_remote_copy(
                    src_ref=gbuf.at[cc], dst_ref=gbuf.at[cc],
                    send_sem=send_sems.at[0], recv_sem=csems.at[cc],
                    device_id=(my,), device_id_type=pl.DeviceIdType.MESH,
                )
                recv.wait_recv()
                out_ref[pl.ds(cc * CHUNK, CHUNK), :] = gbuf[cc].astype(
                    jnp.float32)

        for rdma in sends:
            rdma.wait_send()

    return pl.pallas_call(
        body,
        out_shape=jax.ShapeDtypeStruct((M, N), jnp.float32),
        in_specs=[
            pl.BlockSpec(memory_space=pltpu.VMEM),
            pl.BlockSpec(memory_space=pltpu.VMEM),
        ],
        out_specs=pl.BlockSpec(memory_space=pltpu.VMEM),
        scratch_shapes=[
            pltpu.VMEM((RBLOCK, N), jnp.float32),
            pltpu.VMEM((M, N), jnp.bfloat16),
            pltpu.VMEM((RBLOCK, N), jnp.float32),
            pltpu.VMEM((RBLOCK, N), jnp.bfloat16),
            pltpu.VMEM((PLANE, RBLOCK, N), jnp.bfloat16),
            pltpu.VMEM((ZDIM, CHUNK, N), jnp.bfloat16),
            pltpu.VMEM((N_DEV, CHUNK, N), jnp.bfloat16),
            pltpu.SemaphoreType.DMA((N_SEND,)),
            pltpu.SemaphoreType.DMA((PLANE * ZDIM,)),
            pltpu.SemaphoreType.DMA((ZDIM,)),
            pltpu.SemaphoreType.DMA((N_DEV,)),
        ],
        compiler_params=pltpu.CompilerParams(collective_id=0),
    )(A, B)


# device time: 30978 ns/iter; 1.0227x vs baseline; 1.0227x over previous
import jax
import jax.numpy as jnp
from jax import lax
from jax.experimental import pallas as pl
from jax.experimental.pallas import tpu as pltpu

N_DEV = 16
PLANE = 4
ZDIM = 4
M = 768
N = 768
CHUNK = M // N_DEV
RBLOCK = M // PLANE

N_SEND = 30


def kernel(A, B):
    def body(a_ref, b_ref, out_ref, acc_ref, sbuf, prbuf, prb16, pbuf,
             zbuf, gbuf, send_sems, prsems, zrsems, csems):
        my = lax.axis_index("i")
        z = my // PLANE
        p = my % PLANE

        def plane_peer(off):
            return z * PLANE + (p + off) % PLANE

        def col_peer(off):
            return ((z + off) % ZDIM) * PLANE + p

        barrier_sem = pltpu.get_barrier_semaphore()
        for off in range(1, PLANE):
            pl.semaphore_signal(
                barrier_sem, inc=1, device_id=(plane_peer(off),),
                device_id_type=pl.DeviceIdType.MESH,
            )
        for off in range(1, ZDIM):
            pl.semaphore_signal(
                barrier_sem, inc=1, device_id=(col_peer(off),),
                device_id_type=pl.DeviceIdType.MESH,
            )
        pl.semaphore_wait(barrier_sem, 6)

        acc_ref[...] = jnp.dot(
            a_ref[...].astype(jnp.bfloat16),
            b_ref[...].astype(jnp.bfloat16),
            preferred_element_type=jnp.float32,
        )
        sbuf[...] = acc_ref[...].astype(jnp.bfloat16)

        sends = []
        slot = 0

        zz_order = [1, 2, 3, 0]
        for zoff in zz_order:
            zz = (z + zoff) % ZDIM
            for off in range(1, PLANE):
                pp = (p + off) % PLANE
                rdma = pltpu.make_async_remote_copy(
                    src_ref=sbuf.at[
                        pl.ds(pp * RBLOCK + zz * CHUNK, CHUNK), :],
                    dst_ref=pbuf.at[p, pl.ds(zz * CHUNK, CHUNK), :],
                    send_sem=send_sems.at[slot],
                    recv_sem=prsems.at[p * ZDIM + zz],
                    device_id=(plane_peer(off),),
                    device_id_type=pl.DeviceIdType.MESH,
                )
                rdma.start()
                sends.append(rdma)
                slot += 1

        for zoff in zz_order:
            zz = (z + zoff) % ZDIM
            sub = acc_ref[pl.ds(p * RBLOCK + zz * CHUNK, CHUNK), :]
            for off in range(1, PLANE):
                pp = (p - off) % PLANE
                recv = pltpu.make_async_remote_copy(
                    src_ref=pbuf.at[pp, pl.ds(zz * CHUNK, CHUNK), :],
                    dst_ref=pbuf.at[pp, pl.ds(zz * CHUNK, CHUNK), :],
                    send_sem=send_sems.at[0],
                    recv_sem=prsems.at[pp * ZDIM + zz],
                    device_id=(my,), device_id_type=pl.DeviceIdType.MESH,
                )
                recv.wait_recv()
                sub = sub + pbuf[pp, pl.ds(zz * CHUNK, CHUNK), :].astype(
                    jnp.float32)
            prbuf[pl.ds(zz * CHUNK, CHUNK), :] = sub
            if zoff != 0:
                prb16[pl.ds(zz * CHUNK, CHUNK), :] = sub.astype(jnp.bfloat16)
                rdma = pltpu.make_async_remote_copy(
                    src_ref=prb16.at[pl.ds(zz * CHUNK, CHUNK), :],
                    dst_ref=zbuf.at[z],
                    send_sem=send_sems.at[slot],
                    recv_sem=zrsems.at[z],
                    device_id=(col_peer(zoff),),
                    device_id_type=pl.DeviceIdType.MESH,
                )
                rdma.start()
                sends.append(rdma)
                slot += 1

        red = prbuf[pl.ds(z * CHUNK, CHUNK), :]
        for off in range(1, ZDIM):
            zz = (z - off) % ZDIM
            recv = pltpu.make_async_remote_copy(
                src_ref=zbuf.at[zz], dst_ref=zbuf.at[zz],
                send_sem=send_sems.at[0], recv_sem=zrsems.at[zz],
                device_id=(my,), device_id_type=pl.DeviceIdType.MESH,
            )
            recv.wait_recv()
            red = red + zbuf[zz].astype(jnp.float32)
        c = p * PLANE + z
        out_ref[pl.ds(c * CHUNK, CHUNK), :] = red
        gbuf[c] = red.astype(jnp.bfloat16)

        for off in range(1, ZDIM):
            rdma = pltpu.make_async_remote_copy(
                src_ref=gbuf.at[c], dst_ref=gbuf.at[c],
                send_sem=send_sems.at[slot],
                recv_sem=csems.at[c],
                device_id=(col_peer(off),),
                device_id_type=pl.DeviceIdType.MESH,
            )
            rdma.start()
            sends.append(rdma)
            slot += 1
        for off in range(1, PLANE):
            rdma = pltpu.make_async_remote_copy(
                src_ref=gbuf.at[c], dst_ref=gbuf.at[c],
                send_sem=send_sems.at[slot],
                recv_sem=csems.at[c],
                device_id=(plane_peer(off),),
                device_id_type=pl.DeviceIdType.MESH,
            )
            rdma.start()
            sends.append(rdma)
            slot += 1

        for off in range(1, ZDIM):
            zz = (z - off) % ZDIM
            cc = p * PLANE + zz
            recv = pltpu.make_async_remote_copy(
                src_ref=gbuf.at[cc], dst_ref=gbuf.at[cc],
                send_sem=send_sems.at[0], recv_sem=csems.at[cc],
                device_id=(my,), device_id_type=pl.DeviceIdType.MESH,
            )
            recv.wait_recv()
            for poff in range(1, PLANE):
                rdma = pltpu.make_async_remote_copy(
                    src_ref=gbuf.at[cc], dst_ref=gbuf.at[cc],
                    send_sem=send_sems.at[slot],
                    recv_sem=csems.at[cc],
                    device_id=(plane_peer(poff),),
                    device_id_type=pl.DeviceIdType.MESH,
                )
                rdma.start()
                sends.append(rdma)
                slot += 1
            out_ref[pl.ds(cc * CHUNK, CHUNK), :] = gbuf[cc].astype(
                jnp.float32)

        for off in range(1, PLANE):
            pp = (p - off) % PLANE
            for zz in range(ZDIM):
                cc = pp * PLANE + zz
                recv = pltpu.make_async_remote_copy(
                    src_ref=gbuf.at[cc], dst_ref=gbuf.at[cc],
                    send_sem=send_sems.at[0], recv_sem=csems.at[cc],
                    device_id=(my,), device_id_type=pl.DeviceIdType.MESH,
                )
                recv.wait_recv()
                out_ref[pl.ds(cc * CHUNK, CHUNK), :] = gbuf[cc].astype(
                    jnp.float32)

        for rdma in sends:
            rdma.wait_send()

    return pl.pallas_call(
        body,
        out_shape=jax.ShapeDtypeStruct((M, N), jnp.float32),
        in_specs=[
            pl.BlockSpec(memory_space=pltpu.VMEM),
            pl.BlockSpec(memory_space=pltpu.VMEM),
        ],
        out_specs=pl.BlockSpec(memory_space=pltpu.VMEM),
        scratch_shapes=[
            pltpu.VMEM((M, N), jnp.float32),
            pltpu.VMEM((M, N), jnp.bfloat16),
            pltpu.VMEM((RBLOCK, N), jnp.float32),
            pltpu.VMEM((RBLOCK, N), jnp.bfloat16),
            pltpu.VMEM((PLANE, RBLOCK, N), jnp.bfloat16),
            pltpu.VMEM((ZDIM, CHUNK, N), jnp.bfloat16),
            pltpu.VMEM((N_DEV, CHUNK, N), jnp.bfloat16),
            pltpu.SemaphoreType.DMA((N_SEND,)),
            pltpu.SemaphoreType.DMA((PLANE * ZDIM,)),
            pltpu.SemaphoreType.DMA((ZDIM,)),
            pltpu.SemaphoreType.DMA((N_DEV,)),
        ],
        compiler_params=pltpu.CompilerParams(collective_id=0),
    )(A, B)


# device time: 30466 ns/iter; 1.0399x vs baseline; 1.0168x over previous
import jax
import jax.numpy as jnp
from jax import lax
from jax.experimental import pallas as pl
from jax.experimental.pallas import tpu as pltpu

N_DEV = 16
PLANE = 4
ZDIM = 4
M = 768
N = 768
CHUNK = M // N_DEV
RBLOCK = M // PLANE

N_SEND = 30


def kernel(A, B):
    def body(a_ref, b_ref, out_ref, acc_ref, sbuf, prbuf, prb16, pbuf,
             zbuf, send_sems, prsems, zrsems, csems):
        my = lax.axis_index("i")
        z = my // PLANE
        p = my % PLANE

        def plane_peer(off):
            return z * PLANE + (p + off) % PLANE

        def col_peer(off):
            return ((z + off) % ZDIM) * PLANE + p

        barrier_sem = pltpu.get_barrier_semaphore()
        for off in range(1, PLANE):
            pl.semaphore_signal(
                barrier_sem, inc=1, device_id=(plane_peer(off),),
                device_id_type=pl.DeviceIdType.MESH,
            )
        for off in range(1, ZDIM):
            pl.semaphore_signal(
                barrier_sem, inc=1, device_id=(col_peer(off),),
                device_id_type=pl.DeviceIdType.MESH,
            )
        pl.semaphore_wait(barrier_sem, 6)

        acc_ref[...] = jnp.dot(
            a_ref[...], b_ref[...], preferred_element_type=jnp.float32
        )
        sbuf[...] = acc_ref[...].astype(jnp.bfloat16)

        sends = []
        slot = 0

        zz_order = [1, 2, 3, 0]
        for zoff in zz_order:
            zz = (z + zoff) % ZDIM
            for off in range(1, PLANE):
                pp = (p + off) % PLANE
                rdma = pltpu.make_async_remote_copy(
                    src_ref=sbuf.at[
                        pl.ds(pp * RBLOCK + zz * CHUNK, CHUNK), :],
                    dst_ref=pbuf.at[p, pl.ds(zz * CHUNK, CHUNK), :],
                    send_sem=send_sems.at[slot],
                    recv_sem=prsems.at[p * ZDIM + zz],
                    device_id=(plane_peer(off),),
                    device_id_type=pl.DeviceIdType.MESH,
                )
                rdma.start()
                sends.append(rdma)
                slot += 1

        for zoff in zz_order:
            zz = (z + zoff) % ZDIM
            sub = acc_ref[pl.ds(p * RBLOCK + zz * CHUNK, CHUNK), :]
            for off in range(1, PLANE):
                pp = (p - off) % PLANE
                recv = pltpu.make_async_remote_copy(
                    src_ref=pbuf.at[pp, pl.ds(zz * CHUNK, CHUNK), :],
                    dst_ref=pbuf.at[pp, pl.ds(zz * CHUNK, CHUNK), :],
                    send_sem=send_sems.at[0],
                    recv_sem=prsems.at[pp * ZDIM + zz],
                    device_id=(my,), device_id_type=pl.DeviceIdType.MESH,
                )
                recv.wait_recv()
                sub = sub + pbuf[pp, pl.ds(zz * CHUNK, CHUNK), :].astype(
                    jnp.float32)
            if zoff == 0:
                prbuf[pl.ds(zz * CHUNK, CHUNK), :] = sub
            else:
                prb16[pl.ds(zz * CHUNK, CHUNK), :] = sub.astype(jnp.bfloat16)
                rdma = pltpu.make_async_remote_copy(
                    src_ref=prb16.at[pl.ds(zz * CHUNK, CHUNK), :],
                    dst_ref=zbuf.at[z],
                    send_sem=send_sems.at[slot],
                    recv_sem=zrsems.at[z],
                    device_id=(col_peer(zoff),),
                    device_id_type=pl.DeviceIdType.MESH,
                )
                rdma.start()
                sends.append(rdma)
                slot += 1

        red = prbuf[pl.ds(z * CHUNK, CHUNK), :]
        for off in range(1, ZDIM):
            zz = (z - off) % ZDIM
            recv = pltpu.make_async_remote_copy(
                src_ref=zbuf.at[zz], dst_ref=zbuf.at[zz],
                send_sem=send_sems.at[0], recv_sem=zrsems.at[zz],
                device_id=(my,), device_id_type=pl.DeviceIdType.MESH,
            )
            recv.wait_recv()
            red = red + zbuf[zz].astype(jnp.float32)
        c = p * PLANE + z
        out_ref[pl.ds(c * CHUNK, CHUNK), :] = red.astype(jnp.bfloat16)

        for off in range(1, ZDIM):
            rdma = pltpu.make_async_remote_copy(
                src_ref=out_ref.at[pl.ds(c * CHUNK, CHUNK), :],
                dst_ref=out_ref.at[pl.ds(c * CHUNK, CHUNK), :],
                send_sem=send_sems.at[slot],
                recv_sem=csems.at[c],
                device_id=(col_peer(off),),
                device_id_type=pl.DeviceIdType.MESH,
            )
            rdma.start()
            sends.append(rdma)
            slot += 1
        for off in range(1, PLANE):
            rdma = pltpu.make_async_remote_copy(
                src_ref=out_ref.at[pl.ds(c * CHUNK, CHUNK), :],
                dst_ref=out_ref.at[pl.ds(c * CHUNK, CHUNK), :],
                send_sem=send_sems.at[slot],
                recv_sem=csems.at[c],
                device_id=(plane_peer(off),),
                device_id_type=pl.DeviceIdType.MESH,
            )
            rdma.start()
            sends.append(rdma)
            slot += 1

        for off in range(1, ZDIM):
            zz = (z - off) % ZDIM
            cc = p * PLANE + zz
            recv = pltpu.make_async_remote_copy(
                src_ref=out_ref.at[pl.ds(cc * CHUNK, CHUNK), :],
                dst_ref=out_ref.at[pl.ds(cc * CHUNK, CHUNK), :],
                send_sem=send_sems.at[0], recv_sem=csems.at[cc],
                device_id=(my,), device_id_type=pl.DeviceIdType.MESH,
            )
            recv.wait_recv()
            for poff in range(1, PLANE):
                rdma = pltpu.make_async_remote_copy(
                    src_ref=out_ref.at[pl.ds(cc * CHUNK, CHUNK), :],
                    dst_ref=out_ref.at[pl.ds(cc * CHUNK, CHUNK), :],
                    send_sem=send_sems.at[slot],
                    recv_sem=csems.at[cc],
                    device_id=(plane_peer(poff),),
                    device_id_type=pl.DeviceIdType.MESH,
                )
                rdma.start()
                sends.append(rdma)
                slot += 1

        for off in range(1, PLANE):
            pp = (p - off) % PLANE
            for zz in range(ZDIM):
                cc = pp * PLANE + zz
                recv = pltpu.make_async_remote_copy(
                    src_ref=out_ref.at[pl.ds(cc * CHUNK, CHUNK), :],
                    dst_ref=out_ref.at[pl.ds(cc * CHUNK, CHUNK), :],
                    send_sem=send_sems.at[0], recv_sem=csems.at[cc],
                    device_id=(my,), device_id_type=pl.DeviceIdType.MESH,
                )
                recv.wait_recv()

        for rdma in sends:
            rdma.wait_send()

    return pl.pallas_call(
        body,
        out_shape=jax.ShapeDtypeStruct((M, N), jnp.bfloat16),
        in_specs=[
            pl.BlockSpec(memory_space=pltpu.VMEM),
            pl.BlockSpec(memory_space=pltpu.VMEM),
        ],
        out_specs=pl.BlockSpec(memory_space=pltpu.VMEM),
        scratch_shapes=[
            pltpu.VMEM((M, N), jnp.float32),
            pltpu.VMEM((M, N), jnp.bfloat16),
            pltpu.VMEM((RBLOCK, N), jnp.float32),
            pltpu.VMEM((RBLOCK, N), jnp.bfloat16),
            pltpu.VMEM((PLANE, RBLOCK, N), jnp.bfloat16),
            pltpu.VMEM((ZDIM, CHUNK, N), jnp.bfloat16),
            pltpu.SemaphoreType.DMA((N_SEND,)),
            pltpu.SemaphoreType.DMA((PLANE * ZDIM,)),
            pltpu.SemaphoreType.DMA((ZDIM,)),
            pltpu.SemaphoreType.DMA((N_DEV,)),
        ],
        compiler_params=pltpu.CompilerParams(collective_id=0),
    )(A, B)
